# baseline (speedup 1.0000x reference)
import functools

import jax
import jax.numpy as jnp
from jax import lax
from jax.experimental import pallas as pl
from jax.experimental.pallas import tpu as pltpu
from jax.experimental.pallas import tpu_sc as plsc

_CH = 16


def _make_sc_kernel(B, S, H, P):
    info = plsc.get_sparse_core_info()
    NC, NS = info.num_cores, info.num_subcores
    NW = NC * NS
    WPB = NW // B
    REG = 248
    NCH = 16
    SC0 = 128
    LG = 128
    mesh = plsc.VectorSubcoreMesh(core_axis_name="c", subcore_axis_name="s")

    @functools.partial(
        pl.kernel,
        out_type=jax.ShapeDtypeStruct((B, P + S, H), jnp.float32),
        mesh=mesh,
        scratch_types=[
            pltpu.VMEM((_CH, H), jnp.float32),
            pltpu.VMEM((_CH, H), jnp.float32),
        ],
    )
    def sc_copy(p_hbm, x_hbm, o_hbm, buf0, buf1):
        wid = lax.axis_index("s") * NC + lax.axis_index("c")
        b = wid // WPB
        u = wid % WPB
        out0 = SC0 + u * REG
        w0 = out0 - 104

        v = lax.rem(u, 6)
        pltpu.sync_copy(p_hbm.at[pl.ds(v * _CH, _CH)], buf0)
        pltpu.sync_copy(buf0, o_hbm.at[b, pl.ds(v * _CH, _CH)])

        def win(k):
            return x_hbm.at[b, pl.ds(w0 + _CH * k, _CH)]

        def out_slice(k, rows):
            return o_hbm.at[b, pl.ds(out0 + _CH * k, rows)]

        def shift(dst, src, nrows, cross):
            def row_body(r, acc):
                for c in range(LG):
                    dst[r, pl.ds(c * 16, 16)] = dst[r + 4, pl.ds(c * 16, 16)]
                return acc

            lax.fori_loop(0, nrows, row_body, 0, unroll=False)
            if cross:
                def row_body2(r, acc):
                    for c in range(LG):
                        dst[r + 12, pl.ds(c * 16, 16)] = src[r, pl.ds(c * 16, 16)]
                    return acc

                lax.fori_loop(0, 4, row_body2, 0, unroll=False)

        pltpu.sync_copy(win(0), buf0)

        def group(g, acc):
            k0 = 2 * g
            pltpu.sync_copy(win(k0 + 1), buf1)
            shift(buf0, buf1, 12, True)
            pltpu.sync_copy(buf0, out_slice(k0, _CH))
            pltpu.sync_copy(win(k0 + 2), buf0)
            shift(buf1, buf0, 12, True)
            pltpu.sync_copy(buf1, out_slice(k0 + 1, _CH))
            return acc

        lax.fori_loop(0, (NCH - 2) // 2, group, 0, unroll=False)

        pltpu.sync_copy(win(NCH - 1), buf1)
        shift(buf0, buf1, 12, True)
        pltpu.sync_copy(buf0, out_slice(NCH - 2, _CH))
        shift(buf1, buf0, 8, False)
        pltpu.sync_copy(buf1.at[pl.ds(0, 8)], out_slice(NCH - 1, 8))

    return sc_copy


def _fixup_body(o_in_ref, p_ref, xa_ref, xb_ref, out_ref):
    del o_in_ref
    j = pl.program_id(1)

    @pl.when(j == 0)
    def _():
        out_ref[0, :4] = p_ref[96:100]
        out_ref[0, 4:] = xa_ref[0, :28]

    @pl.when(j == 1)
    def _():
        out_ref[0, :4] = xa_ref[0, 28:]
        out_ref[0, 4:] = xb_ref[0, :28]

    @pl.when(j == 2)
    def _():
        out_ref[0, :4] = xb_ref[0, 28:]


def kernel(input_embeddings, prompt_weight):
    B, S, H = input_embeddings.shape
    P = prompt_weight.shape[0]
    sc = _make_sc_kernel(B, S, H, P)
    main = sc(prompt_weight, input_embeddings)

    def oidx(b, j):
        return (b, jnp.where(j == 0, 3, jnp.where(j == 1, 66, 67)), 0)

    def xaidx(b, j):
        return (b, jnp.where(j == 0, 0, jnp.where(j == 1, 62, 63)), 0)

    def xbidx(b, j):
        return (b, jnp.where(j == 0, 0, 63), 0)

    return pl.pallas_call(
        _fixup_body,
        grid=(B, 3),
        out_shape=jax.ShapeDtypeStruct((B, P + S, H), input_embeddings.dtype),
        in_specs=[
            pl.BlockSpec(memory_space=pltpu.MemorySpace.HBM),
            pl.BlockSpec((P, H), lambda b, j: (0, 0)),
            pl.BlockSpec((1, 32, H), xaidx),
            pl.BlockSpec((1, 32, H), xbidx),
        ],
        out_specs=pl.BlockSpec((1, 32, H), oidx),
        input_output_aliases={0: 0},
    )(main, prompt_weight, input_embeddings, input_embeddings)
